# baseline (speedup 1.0000x reference)
import functools

import jax
import jax.numpy as jnp
from jax import lax
from jax.experimental import pallas as pl
from jax.experimental.pallas import tpu as pltpu, tpu_sc as plsc

_CHUNK = 100
_FIRE = 8
_B_BLK = 512


def _make_sc_pool(n_batch, seq_len, emb, n_workers):
    per_w_b = n_batch // n_workers
    n_chunks = per_w_b * seq_len // _CHUNK
    n_waves = n_chunks // _FIRE
    wave_tok = _FIRE * _CHUNK
    rows_per_wave = wave_tok // seq_len
    mesh = plsc.VectorSubcoreMesh(core_axis_name="c", subcore_axis_name="s")

    @functools.partial(
        pl.kernel,
        out_type=jax.ShapeDtypeStruct((n_batch, emb), jnp.float32),
        mesh=mesh,
        scratch_types=[
            pltpu.VMEM((n_chunks, _CHUNK), jnp.int32),
            pltpu.VMEM((wave_tok, emb), jnp.float32),
            pltpu.VMEM((per_w_b, emb), jnp.float32),
            pltpu.SemaphoreType.DMA,
        ],
        compiler_params=pltpu.CompilerParams(use_tc_tiling_on_sc=False),
    )
    def sc_pool(idx_hbm, table_hbm, out_hbm, idx_v, buf_v, out_v, sem):
        nc = mesh.num_cores
        wid = lax.axis_index("s") * nc + lax.axis_index("c")

        pltpu.sync_copy(idx_hbm.at[wid], idx_v)

        def wave(w, _):
            copies = []
            for i in range(_FIRE):
                copies.append(
                    pltpu.async_copy(
                        table_hbm.at[idx_v.at[w * _FIRE + i]],
                        buf_v.at[pl.ds(i * _CHUNK, _CHUNK)],
                        sem,
                    )
                )
            for c in copies:
                c.wait()

            def pool_one(r, _):
                base = r * seq_len
                accs = ([buf_v[base, pl.ds(q * 16, 16)] for q in range(4)]
                        + [buf_v[base + 1, pl.ds(q * 16, 16)] for q in range(4)])
                for t in range(2, seq_len, 2):
                    for q in range(4):
                        accs[q] = accs[q] + buf_v[base + t, pl.ds(q * 16, 16)]
                        accs[4 + q] = (accs[4 + q]
                                       + buf_v[base + t + 1, pl.ds(q * 16, 16)])
                b = w * rows_per_wave + r
                for q in range(4):
                    out_v[b, pl.ds(q * 16, 16)] = accs[q] + accs[4 + q]
                return _

            lax.fori_loop(0, rows_per_wave, pool_one, None)
            return _

        lax.fori_loop(0, n_waves, wave, None)

        pltpu.sync_copy(out_v, out_hbm.at[pl.ds(wid * per_w_b, per_w_b)])

    return sc_pool


def _fc_body(inv_l, p_ref, w_ref, b_ref, o_ref):
    o_ref[...] = (
        jnp.dot(p_ref[...], w_ref[...], preferred_element_type=jnp.float32)
        * inv_l
        + b_ref[...]
    )


def _fc(pooled, fc_w, fc_b, inv_l):
    n, e = pooled.shape
    c = fc_w.shape[1]
    return pl.pallas_call(
        functools.partial(_fc_body, inv_l),
        grid=(n // _B_BLK,),
        in_specs=[
            pl.BlockSpec((_B_BLK, e), lambda i: (i, 0)),
            pl.BlockSpec((e, c), lambda i: (0, 0)),
            pl.BlockSpec((1, c), lambda i: (0, 0)),
        ],
        out_specs=pl.BlockSpec((_B_BLK, c), lambda i: (i, 0)),
        out_shape=jax.ShapeDtypeStruct((n, c), jnp.float32),
    )(pooled, fc_w, fc_b)


def kernel(text, emb_table, fc_w, fc_b):
    n_batch, seq_len = text.shape
    v, e = emb_table.shape
    info = plsc.get_sparse_core_info()
    n_workers = info.num_cores * info.num_subcores

    idx = text.astype(jnp.int32).reshape(n_workers, -1, _CHUNK)
    pooled = _make_sc_pool(n_batch, seq_len, e, n_workers)(idx, emb_table)
    return _fc(pooled, fc_w, fc_b[None, :], 1.0 / seq_len)


# speedup vs baseline: 1.0038x; 1.0038x over previous
import functools

import jax
import jax.numpy as jnp
from jax import lax
from jax.experimental import pallas as pl
from jax.experimental.pallas import tpu as pltpu, tpu_sc as plsc

_FIRE = 16
_B_BLK = 512


def _make_sc_pool(n_batch, seq_len, emb, n_workers):
    per_w_b = n_batch // n_workers
    n_waves = per_w_b // _FIRE
    wave_tok = _FIRE * seq_len
    mesh = plsc.VectorSubcoreMesh(core_axis_name="c", subcore_axis_name="s")

    @functools.partial(
        pl.kernel,
        out_type=jax.ShapeDtypeStruct((n_batch, emb), jnp.float32),
        mesh=mesh,
        scratch_types=[
            pltpu.VMEM((per_w_b, seq_len), jnp.int32),
            pltpu.VMEM((wave_tok, emb), jnp.float32),
            pltpu.VMEM((per_w_b, emb), jnp.float32),
            pltpu.SemaphoreType.DMA,
        ],
        compiler_params=pltpu.CompilerParams(use_tc_tiling_on_sc=False),
    )
    def sc_pool(idx_hbm, table_hbm, out_hbm, idx_v, buf_v, out_v, sem):
        nc = mesh.num_cores
        wid = lax.axis_index("s") * nc + lax.axis_index("c")

        pltpu.sync_copy(idx_hbm.at[pl.ds(wid * per_w_b, per_w_b)], idx_v)

        def wave(w, _):
            copies = []
            for i in range(_FIRE):
                copies.append(
                    pltpu.async_copy(
                        table_hbm.at[idx_v.at[w * _FIRE + i]],
                        buf_v.at[pl.ds(i * seq_len, seq_len)],
                        sem,
                    )
                )
            for c in copies:
                c.wait()

            def pool_one(r, _):
                base = r * seq_len
                accs = ([buf_v[base, pl.ds(q * 16, 16)] for q in range(4)]
                        + [buf_v[base + 1, pl.ds(q * 16, 16)] for q in range(4)])
                for t in range(2, seq_len, 2):
                    for q in range(4):
                        accs[q] = accs[q] + buf_v[base + t, pl.ds(q * 16, 16)]
                        accs[4 + q] = (accs[4 + q]
                                       + buf_v[base + t + 1, pl.ds(q * 16, 16)])
                b = w * _FIRE + r
                for q in range(4):
                    out_v[b, pl.ds(q * 16, 16)] = accs[q] + accs[4 + q]
                return _

            lax.fori_loop(0, _FIRE, pool_one, None)
            return _

        lax.fori_loop(0, n_waves, wave, None)

        pltpu.sync_copy(out_v, out_hbm.at[pl.ds(wid * per_w_b, per_w_b)])

    return sc_pool


def _fc_body(inv_l, p_ref, w_ref, b_ref, o_ref):
    o_ref[...] = (
        jnp.dot(p_ref[...], w_ref[...], preferred_element_type=jnp.float32)
        * inv_l
        + b_ref[...]
    )


def _fc(pooled, fc_w, fc_b, inv_l):
    n, e = pooled.shape
    c = fc_w.shape[1]
    return pl.pallas_call(
        functools.partial(_fc_body, inv_l),
        grid=(n // _B_BLK,),
        in_specs=[
            pl.BlockSpec((_B_BLK, e), lambda i: (i, 0)),
            pl.BlockSpec((e, c), lambda i: (0, 0)),
            pl.BlockSpec((1, c), lambda i: (0, 0)),
        ],
        out_specs=pl.BlockSpec((_B_BLK, c), lambda i: (i, 0)),
        out_shape=jax.ShapeDtypeStruct((n, c), jnp.float32),
    )(pooled, fc_w, fc_b)


def kernel(text, emb_table, fc_w, fc_b):
    n_batch, seq_len = text.shape
    v, e = emb_table.shape
    info = plsc.get_sparse_core_info()
    n_workers = info.num_cores * info.num_subcores

    idx = text.astype(jnp.int32)
    pooled = _make_sc_pool(n_batch, seq_len, e, n_workers)(idx, emb_table)
    return _fc(pooled, fc_w, fc_b[None, :], 1.0 / seq_len)


# speedup vs baseline: 1.1940x; 1.1895x over previous
import functools

import jax
import jax.numpy as jnp
from jax import lax
from jax.experimental import pallas as pl
from jax.experimental.pallas import tpu as pltpu, tpu_sc as plsc

_PAD_C = 16
_ROW_BLK = 4000


def _proj_body(inv_l, e_ref, w_ref, b_ref, o_ref):
    r8, _ = e_ref.shape
    q = (
        jnp.dot(e_ref[...], w_ref[...], preferred_element_type=jnp.float32)
        + b_ref[...]
    ) * inv_l
    i0 = lax.broadcasted_iota(jnp.int32, q.shape, 0)
    i1 = lax.broadcasted_iota(jnp.int32, q.shape, 1)
    qm = jnp.where((i0 % 8) == (i1 // _PAD_C), q, 0.0)
    step = pl.program_id(0)
    o_ref[pl.ds(step * (r8 // 8), r8 // 8), :] = (
        qm.reshape(r8 // 8, 8, 128).sum(axis=1))


def _project_table(emb_table, w_tiled, b_tiled, inv_l):
    v, e = emb_table.shape
    grid = v // _ROW_BLK
    return pl.pallas_call(
        functools.partial(_proj_body, inv_l),
        grid=(grid,),
        in_specs=[
            pl.BlockSpec((_ROW_BLK, e), lambda i: (i, 0)),
            pl.BlockSpec((e, 128), lambda i: (0, 0)),
            pl.BlockSpec((1, 128), lambda i: (0, 0)),
        ],
        out_specs=pl.BlockSpec((v // 8, 128), lambda i: (0, 0)),
        out_shape=jax.ShapeDtypeStruct((v // 8, 128), jnp.float32),
    )(emb_table, w_tiled, b_tiled)


def _make_sc_pool(n_batch, seq_len, n_workers):
    per_w_tok = n_batch * seq_len // n_workers
    per_w_b = n_batch // n_workers
    chunk = 128
    n_chunks = per_w_tok // chunk
    fire = 10
    mesh = plsc.VectorSubcoreMesh(core_axis_name="c", subcore_axis_name="s")

    @functools.partial(
        pl.kernel,
        out_type=jax.ShapeDtypeStruct((n_batch, _PAD_C), jnp.float32),
        mesh=mesh,
        scratch_types=[
            pltpu.VMEM((n_chunks, chunk), jnp.int32),
            pltpu.VMEM((per_w_tok, _PAD_C), jnp.float32),
            pltpu.VMEM((per_w_b, _PAD_C), jnp.float32),
            pltpu.SemaphoreType.DMA,
        ],
        compiler_params=pltpu.CompilerParams(use_tc_tiling_on_sc=False),
    )
    def sc_pool(idx_hbm, p_hbm, out_hbm, idx_v, rows_v, out_v, sem):
        nc = mesh.num_cores
        wid = lax.axis_index("s") * nc + lax.axis_index("c")

        pltpu.sync_copy(idx_hbm.at[wid], idx_v)

        def gather_group(g, _):
            base = g * fire
            copies = []
            for i in range(fire):
                j = base + i
                copies.append(
                    pltpu.async_copy(
                        p_hbm.at[idx_v.at[j]],
                        rows_v.at[pl.ds(j * chunk, chunk)],
                        sem,
                    )
                )
            for c in copies:
                c.wait()
            return _

        lax.fori_loop(0, n_chunks // fire, gather_group, None)

        def pool_one(b, _):
            base = b * seq_len
            acc0 = rows_v[base]
            acc1 = rows_v[base + 1]
            for t in range(2, seq_len, 2):
                acc0 = acc0 + rows_v[base + t]
                acc1 = acc1 + rows_v[base + t + 1]
            out_v[b] = acc0 + acc1
            return _

        lax.fori_loop(0, per_w_b, pool_one, None)

        pltpu.sync_copy(out_v, out_hbm.at[pl.ds(wid * per_w_b, per_w_b)])

    return sc_pool


def kernel(text, emb_table, fc_w, fc_b):
    n_batch, seq_len = text.shape
    v, e = emb_table.shape
    c = fc_w.shape[1]
    info = plsc.get_sparse_core_info()
    n_workers = info.num_cores * info.num_subcores

    w_pad = jnp.zeros((e, _PAD_C), jnp.float32).at[:, :c].set(fc_w)
    b_pad = jnp.zeros((1, _PAD_C), jnp.float32).at[0, :c].set(fc_b)
    w_tiled = jnp.tile(w_pad, (1, 128 // _PAD_C))
    b_tiled = jnp.tile(b_pad, (1, 128 // _PAD_C))
    proj = _project_table(emb_table, w_tiled, b_tiled, 1.0 / seq_len)
    proj16 = proj.reshape(v, _PAD_C)

    idx = text.astype(jnp.int32).reshape(n_workers, -1, 128)
    pooled = _make_sc_pool(n_batch, seq_len, n_workers)(idx, proj16)
    return pooled[:, :c]


# speedup vs baseline: 1.7592x; 1.4733x over previous
import functools

import jax
import jax.numpy as jnp
from jax import lax
from jax.experimental import pallas as pl
from jax.experimental.pallas import tpu as pltpu, tpu_sc as plsc

_PAD_C = 16
_ROW_BLK = 4096


def _proj_body(inv_l, e_ref, w_ref, b_ref, o_ref):
    _, r8 = e_ref.shape
    q = (
        lax.dot_general(
            e_ref[...], w_ref[...],
            (((0,), (0,)), ((), ())),
            preferred_element_type=jnp.float32,
        )
        + b_ref[...]
    ) * inv_l
    i0 = lax.broadcasted_iota(jnp.int32, q.shape, 0)
    i1 = lax.broadcasted_iota(jnp.int32, q.shape, 1)
    qm = jnp.where((i0 % 8) == (i1 // _PAD_C), q, 0.0)
    o_ref[...] = qm.reshape(r8 // 8, 8, 128).sum(axis=1)


def _project_table(emb_t, w_tiled, b_tiled, inv_l):
    e, v = emb_t.shape
    grid = (v + _ROW_BLK - 1) // _ROW_BLK
    return pl.pallas_call(
        functools.partial(_proj_body, inv_l),
        grid=(grid,),
        in_specs=[
            pl.BlockSpec((e, _ROW_BLK), lambda i: (0, i)),
            pl.BlockSpec((e, 128), lambda i: (0, 0)),
            pl.BlockSpec((1, 128), lambda i: (0, 0)),
        ],
        out_specs=pl.BlockSpec((_ROW_BLK // 8, 128), lambda i: (i, 0)),
        out_shape=jax.ShapeDtypeStruct((v // 8, 128), jnp.float32),
    )(emb_t, w_tiled, b_tiled)


def _make_sc_pool(n_batch, seq_len, n_workers):
    per_w_tok = n_batch * seq_len // n_workers
    per_w_b = n_batch // n_workers
    chunk = 128
    n_chunks = per_w_tok // chunk
    fire = 10
    mesh = plsc.VectorSubcoreMesh(core_axis_name="c", subcore_axis_name="s")

    @functools.partial(
        pl.kernel,
        out_type=jax.ShapeDtypeStruct((n_batch, _PAD_C), jnp.float32),
        mesh=mesh,
        scratch_types=[
            pltpu.VMEM((n_chunks, chunk), jnp.int32),
            pltpu.VMEM((per_w_tok, _PAD_C), jnp.float32),
            pltpu.VMEM((per_w_b, _PAD_C), jnp.float32),
            pltpu.SemaphoreType.DMA,
        ],
        compiler_params=pltpu.CompilerParams(use_tc_tiling_on_sc=False),
    )
    def sc_pool(idx_hbm, p_hbm, out_hbm, idx_v, rows_v, out_v, sem):
        nc = mesh.num_cores
        wid = lax.axis_index("s") * nc + lax.axis_index("c")

        pltpu.sync_copy(idx_hbm.at[wid], idx_v)

        def gather_group(g, _):
            base = g * fire
            copies = []
            for i in range(fire):
                j = base + i
                copies.append(
                    pltpu.async_copy(
                        p_hbm.at[idx_v.at[j]],
                        rows_v.at[pl.ds(j * chunk, chunk)],
                        sem,
                    )
                )
            for c in copies:
                c.wait()
            return _

        lax.fori_loop(0, n_chunks // fire, gather_group, None)

        def pool_one(b, _):
            base = b * seq_len
            acc0 = rows_v[base]
            acc1 = rows_v[base + 1]
            for t in range(2, seq_len, 2):
                acc0 = acc0 + rows_v[base + t]
                acc1 = acc1 + rows_v[base + t + 1]
            out_v[b] = acc0 + acc1
            return _

        lax.fori_loop(0, per_w_b, pool_one, None)

        pltpu.sync_copy(out_v, out_hbm.at[pl.ds(wid * per_w_b, per_w_b)])

    return sc_pool


def kernel(text, emb_table, fc_w, fc_b):
    n_batch, seq_len = text.shape
    v, e = emb_table.shape
    c = fc_w.shape[1]
    info = plsc.get_sparse_core_info()
    n_workers = info.num_cores * info.num_subcores

    w_pad = jnp.zeros((e, _PAD_C), jnp.float32).at[:, :c].set(fc_w)
    b_pad = jnp.zeros((1, _PAD_C), jnp.float32).at[0, :c].set(fc_b)
    w_tiled = jnp.tile(w_pad, (1, 128 // _PAD_C))
    b_tiled = jnp.tile(b_pad, (1, 128 // _PAD_C))
    proj = _project_table(emb_table.T, w_tiled, b_tiled, 1.0 / seq_len)
    proj16 = proj.reshape(v, _PAD_C)

    idx = text.astype(jnp.int32).reshape(n_workers, -1, 128)
    pooled = _make_sc_pool(n_batch, seq_len, n_workers)(idx, proj16)
    return pooled[:, :c]
